# Initial kernel scaffold; baseline (speedup 1.0000x reference)
import jax
import jax.numpy as jnp
from jax.experimental import pallas as pl


def kernel(x, edge_index, W1, b1, W2, b2):
    raise NotImplementedError("write your pallas kernel here")


# speedup vs baseline: 13.5302x; 13.5302x over previous
import functools

import jax
import jax.numpy as jnp
from jax import lax
from jax.experimental import pallas as pl
from jax.experimental.pallas import tpu as pltpu
from jax.experimental.pallas import tpu_sc as plsc

NC = 2
NS = 16
NW = NC * NS

_CHUNK = 80


def _sc_mesh():
    return plsc.VectorSubcoreMesh(core_axis_name="c", subcore_axis_name="s")


def _make_deg_kernel(E, NP):
    ew = E // NW
    nch = ew // _CHUNK
    pt = NP // NS

    @functools.partial(
        pl.kernel,
        mesh=_sc_mesh(),
        out_type=jax.ShapeDtypeStruct((NC, NP), jnp.float32),
        scratch_types=[
            pltpu.VMEM((pt,), jnp.float32),
            pltpu.VMEM((_CHUNK,), jnp.float32),
            pltpu.VMEM((1, _CHUNK), jnp.int32),
            pltpu.VMEM_SHARED((NP,), jnp.float32)
        ],
    )
    def deg_kernel(dst_hbm, degp_hbm, zbuf, ones_v, didx, dacc):
        c = lax.axis_index("c")
        s = lax.axis_index("s")
        wid = c * NS + s

        def _zero(i, _):
            zbuf[pl.ds(i * 16, 16)] = jnp.zeros((16,), jnp.float32)
            return 0
        lax.fori_loop(0, pt // 16, _zero, 0)
        for j in range(_CHUNK // 16):
            ones_v[pl.ds(j * 16, 16)] = jnp.ones((16,), jnp.float32)
        pltpu.sync_copy(zbuf, dacc.at[pl.ds(s * pt, pt)])
        plsc.subcore_barrier()

        base = wid * ew

        def _body(i, _):
            off = base + i * _CHUNK
            pltpu.sync_copy(dst_hbm.at[pl.ds(off, _CHUNK)], didx.at[0])
            pltpu.sync_copy(ones_v, dacc.at[didx.at[0]], add=True)
            return 0
        lax.fori_loop(0, nch, _body, 0)

        plsc.subcore_barrier()
        pltpu.sync_copy(dacc.at[pl.ds(s * pt, pt)],
                        degp_hbm.at[c, pl.ds(s * pt, pt)])

    return deg_kernel


def _make_agg_kernel(N, E, D):
    ew = E // NW
    nch = ew // _CHUNK
    rpt = (N // NS) // 8 * 8
    tail = N - rpt * NS
    zr = 208
    nz = rpt // zr

    @functools.partial(
        pl.kernel,
        mesh=_sc_mesh(),
        out_type=jax.ShapeDtypeStruct((NC, N, D), jnp.float32),
        scratch_types=[
            pltpu.VMEM((zr, D), jnp.float32),
            pltpu.VMEM((1, _CHUNK), jnp.int32),
            pltpu.VMEM((1, _CHUNK), jnp.int32),
            pltpu.VMEM((_CHUNK, D), jnp.float32),
            pltpu.VMEM_SHARED((N, D), jnp.float32),
        ],
    )
    def agg_kernel(g_hbm, src_hbm, dst_hbm, part_hbm,
                   zbuf, sidx, didx, rows, acc):
        c = lax.axis_index("c")
        s = lax.axis_index("s")
        wid = c * NS + s

        def _zero(i, _):
            for j in range(D // 16):
                zbuf[i, pl.ds(j * 16, 16)] = jnp.zeros((16,), jnp.float32)
            return 0
        lax.fori_loop(0, zr, _zero, 0)
        for k in range(nz):
            pltpu.sync_copy(zbuf, acc.at[pl.ds(s * rpt + k * zr, zr)])
        if tail:
            @pl.when(s == NS - 1)
            def _zero_tail():
                pltpu.sync_copy(zbuf.at[pl.ds(0, tail)],
                                acc.at[pl.ds(NS * rpt, tail)])
        plsc.subcore_barrier()

        base = wid * ew

        def _body(i, _):
            off = base + i * _CHUNK
            pltpu.sync_copy(src_hbm.at[pl.ds(off, _CHUNK)], sidx.at[0])
            pltpu.sync_copy(g_hbm.at[sidx.at[0]], rows)
            pltpu.sync_copy(dst_hbm.at[pl.ds(off, _CHUNK)], didx.at[0])
            pltpu.sync_copy(rows, acc.at[didx.at[0]], add=True)
            return 0
        lax.fori_loop(0, nch, _body, 0)

        plsc.subcore_barrier()
        pltpu.sync_copy(acc.at[pl.ds(s * rpt, rpt)],
                        part_hbm.at[c, pl.ds(s * rpt, rpt)])
        if tail:
            @pl.when(s == NS - 1)
            def _copy_tail():
                pltpu.sync_copy(acc.at[pl.ds(NS * rpt, tail)],
                                part_hbm.at[c, pl.ds(NS * rpt, tail)])

    return agg_kernel


def _dis_body(degp_ref, out_ref):
    out_ref[...] = lax.rsqrt(degp_ref[0, :] + degp_ref[1, :] + 1.0)


def _scale_mm_body(x_ref, w_ref, dis_ref, out_ref):
    h = jnp.dot(x_ref[...], w_ref[...], preferred_element_type=jnp.float32)
    out_ref[...] = h * dis_ref[...]


def _mid_body(p_ref, g_ref, dis_ref, b_ref, w_ref, out_ref):
    t = (p_ref[0] + p_ref[1] + g_ref[...]) * dis_ref[...] + b_ref[...]
    o = jnp.maximum(t, 0.0)
    out_ref[...] = jnp.dot(
        o, w_ref[...], preferred_element_type=jnp.float32) * dis_ref[...]


def _final_body(p_ref, g_ref, dis_ref, b_ref, out_ref):
    out_ref[...] = ((p_ref[0] + p_ref[1] + g_ref[...]) * dis_ref[...]
                    + b_ref[...])


def kernel(x, edge_index, W1, b1, W2, b2):
    N, D = x.shape
    E = edge_index.shape[1]
    NP = 10240

    src = edge_index[0]
    dst = edge_index[1]

    degp = _make_deg_kernel(E, NP)(dst)
    dis_flat = pl.pallas_call(
        _dis_body,
        out_shape=jax.ShapeDtypeStruct((NP,), jnp.float32),
    )(degp)
    dis = dis_flat[:N].reshape(N, 1)

    BM = 1000
    grid = (N // BM,)
    row_spec = pl.BlockSpec((BM, D), lambda i: (i, 0))
    dis_spec = pl.BlockSpec((BM, 1), lambda i: (i, 0))
    w_spec = pl.BlockSpec((D, D), lambda i: (0, 0))
    b_spec = pl.BlockSpec((1, D), lambda i: (0, 0))
    p_spec = pl.BlockSpec((NC, BM, D), lambda i: (0, i, 0))
    out_sds = jax.ShapeDtypeStruct((N, D), jnp.float32)

    agg = _make_agg_kernel(N, E, D)

    g1 = pl.pallas_call(
        _scale_mm_body, grid=grid,
        in_specs=[row_spec, w_spec, dis_spec],
        out_specs=row_spec, out_shape=out_sds,
    )(x, W1, dis)

    p1 = agg(g1, src, dst)

    g2 = pl.pallas_call(
        _mid_body, grid=grid,
        in_specs=[p_spec, row_spec, dis_spec, b_spec, w_spec],
        out_specs=row_spec, out_shape=out_sds,
    )(p1, g1, dis, b1.reshape(1, D), W2)

    p2 = agg(g2, src, dst)

    out = pl.pallas_call(
        _final_body, grid=grid,
        in_specs=[p_spec, row_spec, dis_spec, b_spec],
        out_specs=row_spec, out_shape=out_sds,
    )(p2, g2, dis, b2.reshape(1, D))

    return out


# speedup vs baseline: 20.9876x; 1.5512x over previous
import functools

import jax
import jax.numpy as jnp
from jax import lax
from jax.experimental import pallas as pl
from jax.experimental.pallas import tpu as pltpu
from jax.experimental.pallas import tpu_sc as plsc

NC = 2
NS = 16
NW = NC * NS

_CK = 40
_K = 4
_ZR = 208

_DCK = 40


def _sc_mesh():
    return plsc.VectorSubcoreMesh(core_axis_name="c", subcore_axis_name="s")


def _make_deg_kernel(ew, NP):
    nch = ew // _DCK
    pt = NP // NS

    @functools.partial(
        pl.kernel,
        mesh=_sc_mesh(),
        out_type=jax.ShapeDtypeStruct((NC, NP), jnp.float32),
        scratch_types=[
            pltpu.VMEM((pt,), jnp.float32),
            pltpu.VMEM((_DCK,), jnp.float32),
            pltpu.VMEM((nch, _DCK), jnp.int32),
            pltpu.VMEM_SHARED((NP,), jnp.float32),
        ],
    )
    def deg_kernel(dstr_hbm, degp_hbm, zbuf, ones_v, didx, dacc):
        c = lax.axis_index("c")
        s = lax.axis_index("s")
        wid = c * NS + s

        pltpu.sync_copy(dstr_hbm.at[wid], didx)

        def _zero(i, _):
            zbuf[pl.ds(i * 16, 16)] = jnp.zeros((16,), jnp.float32)
            return 0
        lax.fori_loop(0, pt // 16, _zero, 0)
        for j in range(_DCK // 16):
            ones_v[pl.ds(j * 16, 16)] = jnp.ones((16,), jnp.float32)
        ones_v[pl.ds(_DCK - 16, 16)] = jnp.ones((16,), jnp.float32)
        pltpu.sync_copy(zbuf, dacc.at[pl.ds(s * pt, pt)])
        plsc.subcore_barrier()

        def _body(ch, _):
            pltpu.sync_copy(ones_v, dacc.at[didx.at[ch]], add=True)
            return 0
        lax.fori_loop(0, nch, _body, 0)

        plsc.subcore_barrier()
        pltpu.sync_copy(dacc.at[pl.ds(s * pt, pt)],
                        degp_hbm.at[c, pl.ds(s * pt, pt)])

    return deg_kernel


def _make_agg_kernel(N, ew, D):
    nch = ew // _CK
    NA = N + 8
    rpt = (N // NS) // 8 * 8
    tail = N - rpt * NS
    nz = rpt // _ZR

    @functools.partial(
        pl.kernel,
        mesh=_sc_mesh(),
        out_type=jax.ShapeDtypeStruct((NC, N, D), jnp.float32),
        scratch_types=[
            pltpu.VMEM((_K, 2, _CK), jnp.int32),
            pltpu.VMEM((_K, _CK, D), jnp.float32),
            pltpu.VMEM_SHARED((NA, D), jnp.float32),
            pltpu.SemaphoreType.DMA((_K,)),
            pltpu.SemaphoreType.DMA((_K,)),
            pltpu.SemaphoreType.DMA((_K,)),
        ],
    )
    def agg_kernel(g_hbm, sd_hbm, zeros_hbm, part_hbm,
                   idxr, rows, acc, isem, gsem, ssem):
        c = lax.axis_index("c")
        s = lax.axis_index("s")
        wid = c * NS + s

        for k in range(nz):
            pltpu.sync_copy(zeros_hbm, acc.at[pl.ds(s * rpt + k * _ZR, _ZR)])
        if tail:
            @pl.when(s == NS - 1)
            def _zero_tail():
                pltpu.sync_copy(zeros_hbm.at[pl.ds(0, tail)],
                                acc.at[pl.ds(NS * rpt, tail)])
        plsc.subcore_barrier()

        def _idx(ch, b):
            return pltpu.make_async_copy(
                sd_hbm.at[wid, ch], idxr.at[b], isem.at[b])

        def _gather(ch, b):
            return pltpu.make_async_copy(
                g_hbm.at[idxr.at[b, 0]], rows.at[b], gsem.at[b])

        def _scatter(ch, b):
            return pltpu.make_async_copy(
                rows.at[b], acc.at[idxr.at[b, 1]], ssem.at[b])

        def _body(g, _):
            c0 = g * _K
            for b in range(_K):
                _idx(c0 + b, b).start()
            for b in range(_K):
                _idx(c0 + b, b).wait()
                _gather(c0 + b, b).start()
            for b in range(_K):
                _gather(c0 + b, b).wait()
                _scatter(c0 + b, b).start(add=True)
            for b in range(_K):
                _scatter(c0 + b, b).wait()
            return 0
        lax.fori_loop(0, nch // _K, _body, 0)

        plsc.subcore_barrier()
        pltpu.sync_copy(acc.at[pl.ds(s * rpt, rpt)],
                        part_hbm.at[c, pl.ds(s * rpt, rpt)])
        if tail:
            @pl.when(s == NS - 1)
            def _copy_tail():
                pltpu.sync_copy(acc.at[pl.ds(NS * rpt, tail)],
                                part_hbm.at[c, pl.ds(NS * rpt, tail)])

    return agg_kernel


def _dis_body(degp_ref, out_ref):
    out_ref[...] = lax.rsqrt(degp_ref[0, :] + degp_ref[1, :] + 1.0)


def _scale_mm_body(x_ref, w_ref, dis_ref, out_ref):
    h = jnp.dot(x_ref[...], w_ref[...], preferred_element_type=jnp.float32)
    out_ref[...] = h * dis_ref[...]


def _mid_body(p_ref, g_ref, dis_ref, b_ref, w_ref, out_ref):
    t = (p_ref[0] + p_ref[1] + g_ref[...]) * dis_ref[...] + b_ref[...]
    o = jnp.maximum(t, 0.0)
    out_ref[...] = jnp.dot(
        o, w_ref[...], preferred_element_type=jnp.float32) * dis_ref[...]


def _final_body(p_ref, g_ref, dis_ref, b_ref, out_ref):
    out_ref[...] = ((p_ref[0] + p_ref[1] + g_ref[...]) * dis_ref[...]
                    + b_ref[...])


def kernel(x, edge_index, W1, b1, W2, b2):
    N, D = x.shape
    E = edge_index.shape[1]
    NP = 10240

    gran = NW * 320
    E_pad = -(-E // gran) * gran
    pad = E_pad - E
    ew = E_pad // NW
    nch = ew // _CK

    src_p = jnp.concatenate(
        [edge_index[0], (jnp.arange(pad, dtype=jnp.int32) % N)])
    dst_p = jnp.concatenate(
        [edge_index[1], N + (jnp.arange(pad, dtype=jnp.int32) % 8)])
    sd = jnp.stack([src_p.reshape(NW, nch, _CK),
                    dst_p.reshape(NW, nch, _CK)], axis=2)
    dst_r = dst_p.reshape(NW, ew // _DCK, _DCK)

    degp = _make_deg_kernel(ew, NP)(dst_r)
    dis_flat = pl.pallas_call(
        _dis_body,
        out_shape=jax.ShapeDtypeStruct((NP,), jnp.float32),
    )(degp)
    dis = dis_flat[:N].reshape(N, 1)

    BM = 1000
    grid = (N // BM,)
    row_spec = pl.BlockSpec((BM, D), lambda i: (i, 0))
    dis_spec = pl.BlockSpec((BM, 1), lambda i: (i, 0))
    w_spec = pl.BlockSpec((D, D), lambda i: (0, 0))
    b_spec = pl.BlockSpec((1, D), lambda i: (0, 0))
    p_spec = pl.BlockSpec((NC, BM, D), lambda i: (0, i, 0))
    out_sds = jax.ShapeDtypeStruct((N, D), jnp.float32)

    agg = _make_agg_kernel(N, ew, D)
    zeros_z = jnp.zeros((_ZR, D), jnp.float32)

    g1 = pl.pallas_call(
        _scale_mm_body, grid=grid,
        in_specs=[row_spec, w_spec, dis_spec],
        out_specs=row_spec, out_shape=out_sds,
    )(x, W1, dis)

    p1 = agg(g1, sd, zeros_z)

    g2 = pl.pallas_call(
        _mid_body, grid=grid,
        in_specs=[p_spec, row_spec, dis_spec, b_spec, w_spec],
        out_specs=row_spec, out_shape=out_sds,
    )(p1, g1, dis, b1.reshape(1, D), W2)

    p2 = agg(g2, sd, zeros_z)

    out = pl.pallas_call(
        _final_body, grid=grid,
        in_specs=[p_spec, row_spec, dis_spec, b_spec],
        out_specs=row_spec, out_shape=out_sds,
    )(p2, g2, dis, b2.reshape(1, D))

    return out


# speedup vs baseline: 23.3339x; 1.1118x over previous
import functools

import jax
import jax.numpy as jnp
from jax import lax
from jax.experimental import pallas as pl
from jax.experimental.pallas import tpu as pltpu
from jax.experimental.pallas import tpu_sc as plsc

NC = 2
NS = 16
NW = NC * NS

_CK = 20
_K = 4
_NB = 2
_ZR = 208

_DCK = 40


def _sc_mesh():
    return plsc.VectorSubcoreMesh(core_axis_name="c", subcore_axis_name="s")


def _make_deg_kernel(ew, NP):
    nch = ew // _DCK
    pt = NP // NS

    @functools.partial(
        pl.kernel,
        mesh=_sc_mesh(),
        out_type=jax.ShapeDtypeStruct((NC, NP), jnp.float32),
        scratch_types=[
            pltpu.VMEM((pt,), jnp.float32),
            pltpu.VMEM((_DCK,), jnp.float32),
            pltpu.VMEM((nch, _DCK), jnp.int32),
            pltpu.VMEM_SHARED((NP,), jnp.float32),
        ],
    )
    def deg_kernel(dstr_hbm, degp_hbm, zbuf, ones_v, didx, dacc):
        c = lax.axis_index("c")
        s = lax.axis_index("s")
        wid = c * NS + s

        pltpu.sync_copy(dstr_hbm.at[wid], didx)

        def _zero(i, _):
            zbuf[pl.ds(i * 16, 16)] = jnp.zeros((16,), jnp.float32)
            return 0
        lax.fori_loop(0, pt // 16, _zero, 0)
        for j in range(_DCK // 16):
            ones_v[pl.ds(j * 16, 16)] = jnp.ones((16,), jnp.float32)
        ones_v[pl.ds(_DCK - 16, 16)] = jnp.ones((16,), jnp.float32)
        pltpu.sync_copy(zbuf, dacc.at[pl.ds(s * pt, pt)])
        plsc.subcore_barrier()

        def _body(ch, _):
            pltpu.sync_copy(ones_v, dacc.at[didx.at[ch]], add=True)
            return 0
        lax.fori_loop(0, nch, _body, 0)

        plsc.subcore_barrier()
        pltpu.sync_copy(dacc.at[pl.ds(s * pt, pt)],
                        degp_hbm.at[c, pl.ds(s * pt, pt)])

    return deg_kernel


def _make_agg_kernel(N, ew, D):
    nch = ew // _CK
    NA = N + 8
    rpt = (N // NS) // 8 * 8
    tail = N - rpt * NS
    nz = rpt // _ZR

    @functools.partial(
        pl.kernel,
        mesh=_sc_mesh(),
        out_type=jax.ShapeDtypeStruct((NC, N, D), jnp.float32),
        scratch_types=[
            pltpu.VMEM((_NB * _K, 2, _CK), jnp.int32),
            pltpu.VMEM((_NB * _K, _CK, D), jnp.float32),
            pltpu.VMEM_SHARED((NA, D), jnp.float32),
            pltpu.SemaphoreType.DMA((_NB * _K,)),
            pltpu.SemaphoreType.DMA((_NB * _K,)),
            pltpu.SemaphoreType.DMA((_NB * _K,)),
        ],
    )
    def agg_kernel(g_hbm, sd_hbm, zeros_hbm, part_hbm,
                   idxr, rows, acc, isem, gsem, ssem):
        c = lax.axis_index("c")
        s = lax.axis_index("s")
        wid = c * NS + s

        for k in range(nz):
            pltpu.sync_copy(zeros_hbm, acc.at[pl.ds(s * rpt + k * _ZR, _ZR)])
        if tail:
            @pl.when(s == NS - 1)
            def _zero_tail():
                pltpu.sync_copy(zeros_hbm.at[pl.ds(0, tail)],
                                acc.at[pl.ds(NS * rpt, tail)])
        plsc.subcore_barrier()

        def _idx(ch, b):
            return pltpu.make_async_copy(
                sd_hbm.at[wid, ch], idxr.at[b], isem.at[b])

        def _gather(b):
            return pltpu.make_async_copy(
                g_hbm.at[idxr.at[b, 0]], rows.at[b], gsem.at[b])

        def _scatter(b):
            return pltpu.make_async_copy(
                rows.at[b], acc.at[idxr.at[b, 1]], ssem.at[b])

        def _fire_idx(g, B):
            for b in range(_K):
                _idx(g * _K + b, B + b).start()

        def _fire_gather(g, B):
            for b in range(_K):
                _idx(g * _K + b, B + b).wait()
                _gather(B + b).start()

        def _fire_scatter(B):
            for b in range(_K):
                _gather(B + b).wait()
                _scatter(B + b).start(add=True)

        def _drain_scatter(B):
            for b in range(_K):
                _scatter(B + b).wait()

        ngrp = nch // _K

        _fire_idx(0, 0)
        _fire_gather(0, 0)
        _fire_idx(1, _K)
        _fire_scatter(0)

        def _step(g, B, Bo):
            _fire_gather(g, B)
            _drain_scatter(Bo)
            _fire_idx(g + 1, Bo)
            _fire_scatter(B)

        def _body(p, _):
            _step(2 * p + 1, _K, 0)
            _step(2 * p + 2, 0, _K)
            return 0
        lax.fori_loop(0, (ngrp - 2) // 2, _body, 0)

        _fire_gather(ngrp - 1, _K)
        _drain_scatter(0)
        _fire_scatter(_K)
        _drain_scatter(_K)

        plsc.subcore_barrier()
        pltpu.sync_copy(acc.at[pl.ds(s * rpt, rpt)],
                        part_hbm.at[c, pl.ds(s * rpt, rpt)])
        if tail:
            @pl.when(s == NS - 1)
            def _copy_tail():
                pltpu.sync_copy(acc.at[pl.ds(NS * rpt, tail)],
                                part_hbm.at[c, pl.ds(NS * rpt, tail)])

    return agg_kernel


def _dis_body(degp_ref, out_ref):
    out_ref[...] = lax.rsqrt(degp_ref[0, :] + degp_ref[1, :] + 1.0)


def _scale_mm_body(x_ref, w_ref, dis_ref, out_ref):
    h = jnp.dot(x_ref[...], w_ref[...], preferred_element_type=jnp.float32)
    out_ref[...] = h * dis_ref[...]


def _mid_body(p_ref, g_ref, dis_ref, b_ref, w_ref, out_ref):
    t = (p_ref[0] + p_ref[1] + g_ref[...]) * dis_ref[...] + b_ref[...]
    o = jnp.maximum(t, 0.0)
    out_ref[...] = jnp.dot(
        o, w_ref[...], preferred_element_type=jnp.float32) * dis_ref[...]


def _final_body(p_ref, g_ref, dis_ref, b_ref, out_ref):
    out_ref[...] = ((p_ref[0] + p_ref[1] + g_ref[...]) * dis_ref[...]
                    + b_ref[...])


def kernel(x, edge_index, W1, b1, W2, b2):
    N, D = x.shape
    E = edge_index.shape[1]
    NP = 10240

    gran = NW * 320
    E_pad = -(-E // gran) * gran
    pad = E_pad - E
    ew = E_pad // NW
    nch = ew // _CK

    src_p = jnp.concatenate(
        [edge_index[0], (jnp.arange(pad, dtype=jnp.int32) % N)])
    dst_p = jnp.concatenate(
        [edge_index[1], N + (jnp.arange(pad, dtype=jnp.int32) % 8)])
    sd = jnp.stack([src_p.reshape(NW, nch, _CK),
                    dst_p.reshape(NW, nch, _CK)], axis=2)
    dst_r = dst_p.reshape(NW, ew // _DCK, _DCK)

    degp = _make_deg_kernel(ew, NP)(dst_r)
    dis_flat = pl.pallas_call(
        _dis_body,
        out_shape=jax.ShapeDtypeStruct((NP,), jnp.float32),
    )(degp)
    dis = dis_flat[:N].reshape(N, 1)

    BM = 1000
    grid = (N // BM,)
    row_spec = pl.BlockSpec((BM, D), lambda i: (i, 0))
    dis_spec = pl.BlockSpec((BM, 1), lambda i: (i, 0))
    w_spec = pl.BlockSpec((D, D), lambda i: (0, 0))
    b_spec = pl.BlockSpec((1, D), lambda i: (0, 0))
    p_spec = pl.BlockSpec((NC, BM, D), lambda i: (0, i, 0))
    out_sds = jax.ShapeDtypeStruct((N, D), jnp.float32)

    agg = _make_agg_kernel(N, ew, D)
    zeros_z = jnp.zeros((_ZR, D), jnp.float32)

    g1 = pl.pallas_call(
        _scale_mm_body, grid=grid,
        in_specs=[row_spec, w_spec, dis_spec],
        out_specs=row_spec, out_shape=out_sds,
    )(x, W1, dis)

    p1 = agg(g1, sd, zeros_z)

    g2 = pl.pallas_call(
        _mid_body, grid=grid,
        in_specs=[p_spec, row_spec, dis_spec, b_spec, w_spec],
        out_specs=row_spec, out_shape=out_sds,
    )(p1, g1, dis, b1.reshape(1, D), W2)

    p2 = agg(g2, sd, zeros_z)

    out = pl.pallas_call(
        _final_body, grid=grid,
        in_specs=[p_spec, row_spec, dis_spec, b_spec],
        out_specs=row_spec, out_shape=out_sds,
    )(p2, g2, dis, b2.reshape(1, D))

    return out


# speedup vs baseline: 24.7507x; 1.0607x over previous
import functools

import jax
import jax.numpy as jnp
from jax import lax
from jax.experimental import pallas as pl
from jax.experimental.pallas import tpu as pltpu
from jax.experimental.pallas import tpu_sc as plsc

NC = 2
NS = 16
NW = NC * NS

_CK = 40
_K = 2
_NB = 2
_ZR = 208

_DCK = 40


def _sc_mesh():
    return plsc.VectorSubcoreMesh(core_axis_name="c", subcore_axis_name="s")


def _make_deg_kernel(ew, NP):
    nch = ew // _DCK
    pt = NP // NS

    @functools.partial(
        pl.kernel,
        mesh=_sc_mesh(),
        out_type=jax.ShapeDtypeStruct((NC, NP), jnp.float32),
        scratch_types=[
            pltpu.VMEM((pt,), jnp.float32),
            pltpu.VMEM((_DCK,), jnp.float32),
            pltpu.VMEM((nch, _DCK), jnp.int32),
            pltpu.VMEM_SHARED((NP,), jnp.float32),
        ],
    )
    def deg_kernel(dstr_hbm, degp_hbm, zbuf, ones_v, didx, dacc):
        c = lax.axis_index("c")
        s = lax.axis_index("s")
        wid = c * NS + s

        pltpu.sync_copy(dstr_hbm.at[wid], didx)

        def _zero(i, _):
            zbuf[pl.ds(i * 16, 16)] = jnp.zeros((16,), jnp.float32)
            return 0
        lax.fori_loop(0, pt // 16, _zero, 0)
        for j in range(_DCK // 16):
            ones_v[pl.ds(j * 16, 16)] = jnp.ones((16,), jnp.float32)
        ones_v[pl.ds(_DCK - 16, 16)] = jnp.ones((16,), jnp.float32)
        pltpu.sync_copy(zbuf, dacc.at[pl.ds(s * pt, pt)])
        plsc.subcore_barrier()

        def _body(ch, _):
            pltpu.sync_copy(ones_v, dacc.at[didx.at[ch]], add=True)
            return 0
        lax.fori_loop(0, nch, _body, 0)

        plsc.subcore_barrier()
        pltpu.sync_copy(dacc.at[pl.ds(s * pt, pt)],
                        degp_hbm.at[c, pl.ds(s * pt, pt)])

    return deg_kernel


def _make_agg_kernel(N, ew, D):
    nch = ew // _CK
    NA = N + 8
    rpt = (N // NS) // 8 * 8
    tail = N - rpt * NS
    nz = rpt // _ZR

    @functools.partial(
        pl.kernel,
        mesh=_sc_mesh(),
        out_type=jax.ShapeDtypeStruct((NC, N, D), jnp.float32),
        scratch_types=[
            pltpu.VMEM((_NB * _K, 2, _CK), jnp.int32),
            pltpu.VMEM((_NB * _K, _CK, D), jnp.float32),
            pltpu.VMEM_SHARED((NA, D), jnp.float32),
            pltpu.SemaphoreType.DMA((_NB * _K,)),
            pltpu.SemaphoreType.DMA((_NB * _K,)),
            pltpu.SemaphoreType.DMA((_NB * _K,)),
        ],
    )
    def agg_kernel(g_hbm, sd_hbm, zeros_hbm, part_hbm,
                   idxr, rows, acc, isem, gsem, ssem):
        c = lax.axis_index("c")
        s = lax.axis_index("s")
        wid = c * NS + s

        for k in range(nz):
            pltpu.sync_copy(zeros_hbm, acc.at[pl.ds(s * rpt + k * _ZR, _ZR)])
        if tail:
            @pl.when(s == NS - 1)
            def _zero_tail():
                pltpu.sync_copy(zeros_hbm.at[pl.ds(0, tail)],
                                acc.at[pl.ds(NS * rpt, tail)])
        plsc.subcore_barrier()

        def _idx(ch, b):
            return pltpu.make_async_copy(
                sd_hbm.at[wid, ch], idxr.at[b], isem.at[b])

        def _gather(b):
            return pltpu.make_async_copy(
                g_hbm.at[idxr.at[b, 0]], rows.at[b], gsem.at[b])

        def _scatter(b):
            return pltpu.make_async_copy(
                rows.at[b], acc.at[idxr.at[b, 1]], ssem.at[b])

        def _fire_idx(g, B):
            for b in range(_K):
                _idx(g * _K + b, B + b).start()

        def _fire_gather(g, B):
            for b in range(_K):
                _idx(g * _K + b, B + b).wait()
                _gather(B + b).start()

        def _fire_scatter(B):
            for b in range(_K):
                _gather(B + b).wait()
                _scatter(B + b).start(add=True)

        def _drain_scatter(B):
            for b in range(_K):
                _scatter(B + b).wait()

        ngrp = nch // _K

        _fire_idx(0, 0)
        _fire_gather(0, 0)
        _fire_idx(1, _K)
        _fire_scatter(0)

        def _step(g, B, Bo):
            _fire_gather(g, B)
            _drain_scatter(Bo)
            _fire_idx(g + 1, Bo)
            _fire_scatter(B)

        def _body(p, _):
            _step(2 * p + 1, _K, 0)
            _step(2 * p + 2, 0, _K)
            return 0
        lax.fori_loop(0, (ngrp - 2) // 2, _body, 0)

        _fire_gather(ngrp - 1, _K)
        _drain_scatter(0)
        _fire_scatter(_K)
        _drain_scatter(_K)

        plsc.subcore_barrier()
        pltpu.sync_copy(acc.at[pl.ds(s * rpt, rpt)],
                        part_hbm.at[c, pl.ds(s * rpt, rpt)])
        if tail:
            @pl.when(s == NS - 1)
            def _copy_tail():
                pltpu.sync_copy(acc.at[pl.ds(NS * rpt, tail)],
                                part_hbm.at[c, pl.ds(NS * rpt, tail)])

    return agg_kernel


def _dis_body(degp_ref, out_ref):
    out_ref[...] = lax.rsqrt(degp_ref[0, :] + degp_ref[1, :] + 1.0)


def _scale_mm_body(x_ref, w_ref, dis_ref, out_ref):
    h = jnp.dot(x_ref[...], w_ref[...], preferred_element_type=jnp.float32)
    out_ref[...] = h * dis_ref[...]


def _mid_body(p_ref, g_ref, dis_ref, b_ref, w_ref, out_ref):
    t = (p_ref[0] + p_ref[1] + g_ref[...]) * dis_ref[...] + b_ref[...]
    o = jnp.maximum(t, 0.0)
    out_ref[...] = jnp.dot(
        o, w_ref[...], preferred_element_type=jnp.float32) * dis_ref[...]


def _final_body(p_ref, g_ref, dis_ref, b_ref, out_ref):
    out_ref[...] = ((p_ref[0] + p_ref[1] + g_ref[...]) * dis_ref[...]
                    + b_ref[...])


def kernel(x, edge_index, W1, b1, W2, b2):
    N, D = x.shape
    E = edge_index.shape[1]
    NP = 10240

    gran = NW * 320
    E_pad = -(-E // gran) * gran
    pad = E_pad - E
    ew = E_pad // NW
    nch = ew // _CK

    src_p = jnp.concatenate(
        [edge_index[0], (jnp.arange(pad, dtype=jnp.int32) % N)])
    dst_p = jnp.concatenate(
        [edge_index[1], N + (jnp.arange(pad, dtype=jnp.int32) % 8)])
    sd = jnp.stack([src_p.reshape(NW, nch, _CK),
                    dst_p.reshape(NW, nch, _CK)], axis=2)
    dst_r = dst_p.reshape(NW, ew // _DCK, _DCK)

    degp = _make_deg_kernel(ew, NP)(dst_r)
    dis_flat = pl.pallas_call(
        _dis_body,
        out_shape=jax.ShapeDtypeStruct((NP,), jnp.float32),
    )(degp)
    dis = dis_flat[:N].reshape(N, 1)

    BM = 1000
    grid = (N // BM,)
    row_spec = pl.BlockSpec((BM, D), lambda i: (i, 0))
    dis_spec = pl.BlockSpec((BM, 1), lambda i: (i, 0))
    w_spec = pl.BlockSpec((D, D), lambda i: (0, 0))
    b_spec = pl.BlockSpec((1, D), lambda i: (0, 0))
    p_spec = pl.BlockSpec((NC, BM, D), lambda i: (0, i, 0))
    out_sds = jax.ShapeDtypeStruct((N, D), jnp.float32)

    agg = _make_agg_kernel(N, ew, D)
    zeros_z = jnp.zeros((_ZR, D), jnp.float32)

    g1 = pl.pallas_call(
        _scale_mm_body, grid=grid,
        in_specs=[row_spec, w_spec, dis_spec],
        out_specs=row_spec, out_shape=out_sds,
    )(x, W1, dis)

    p1 = agg(g1, sd, zeros_z)

    g2 = pl.pallas_call(
        _mid_body, grid=grid,
        in_specs=[p_spec, row_spec, dis_spec, b_spec, w_spec],
        out_specs=row_spec, out_shape=out_sds,
    )(p1, g1, dis, b1.reshape(1, D), W2)

    p2 = agg(g2, sd, zeros_z)

    out = pl.pallas_call(
        _final_body, grid=grid,
        in_specs=[p_spec, row_spec, dis_spec, b_spec],
        out_specs=row_spec, out_shape=out_sds,
    )(p2, g2, dis, b2.reshape(1, D))

    return out


# speedup vs baseline: 24.9822x; 1.0094x over previous
import functools

import jax
import jax.numpy as jnp
from jax import lax
from jax.experimental import pallas as pl
from jax.experimental.pallas import tpu as pltpu
from jax.experimental.pallas import tpu_sc as plsc

NC = 2
NS = 16
NW = NC * NS

_CK = 40
_K = 2
_NB = 2
_ZR = 208

_DCK = 40


def _sc_mesh():
    return plsc.VectorSubcoreMesh(core_axis_name="c", subcore_axis_name="s")


def _make_deg_kernel(ew, NP):
    nch = ew // _DCK
    pt = NP // NS

    @functools.partial(
        pl.kernel,
        mesh=_sc_mesh(),
        out_type=jax.ShapeDtypeStruct((NC, NP), jnp.float32),
        scratch_types=[
            pltpu.VMEM((pt,), jnp.float32),
            pltpu.VMEM((_DCK,), jnp.float32),
            pltpu.VMEM((nch, _DCK), jnp.int32),
            pltpu.VMEM_SHARED((NP,), jnp.float32),
            pltpu.SemaphoreType.DMA((4,)),
        ],
    )
    def deg_kernel(dstr_hbm, degp_hbm, zbuf, ones_v, didx, dacc, dsem):
        c = lax.axis_index("c")
        s = lax.axis_index("s")
        wid = c * NS + s

        pltpu.sync_copy(dstr_hbm.at[wid], didx)

        def _zero(i, _):
            zbuf[pl.ds(i * 16, 16)] = jnp.zeros((16,), jnp.float32)
            return 0
        lax.fori_loop(0, pt // 16, _zero, 0)
        for j in range(_DCK // 16):
            ones_v[pl.ds(j * 16, 16)] = jnp.ones((16,), jnp.float32)
        ones_v[pl.ds(_DCK - 16, 16)] = jnp.ones((16,), jnp.float32)
        pltpu.sync_copy(zbuf, dacc.at[pl.ds(s * pt, pt)])
        plsc.subcore_barrier()

        def _sc_add(ch, b):
            return pltpu.make_async_copy(
                ones_v, dacc.at[didx.at[ch]], dsem.at[b])

        def _body(g, _):
            c0 = g * 4
            for b in range(4):
                _sc_add(c0 + b, b).start(add=True)
            for b in range(4):
                _sc_add(c0 + b, b).wait()
            return 0
        lax.fori_loop(0, nch // 4, _body, 0)

        plsc.subcore_barrier()
        pltpu.sync_copy(dacc.at[pl.ds(s * pt, pt)],
                        degp_hbm.at[c, pl.ds(s * pt, pt)])

    return deg_kernel


def _make_agg_kernel(N, ew, D):
    nch = ew // _CK
    NA = N + 8
    rpt = (N // NS) // 8 * 8
    tail = N - rpt * NS
    nz = rpt // _ZR

    @functools.partial(
        pl.kernel,
        mesh=_sc_mesh(),
        out_type=jax.ShapeDtypeStruct((NC, N, D), jnp.float32),
        scratch_types=[
            pltpu.VMEM((_NB * _K, 2, _CK), jnp.int32),
            pltpu.VMEM((_NB * _K, _CK, D), jnp.float32),
            pltpu.VMEM_SHARED((NA, D), jnp.float32),
            pltpu.SemaphoreType.DMA((_NB * _K,)),
            pltpu.SemaphoreType.DMA((_NB * _K,)),
            pltpu.SemaphoreType.DMA((_NB * _K,)),
        ],
    )
    def agg_kernel(g_hbm, sd_hbm, zeros_hbm, part_hbm,
                   idxr, rows, acc, isem, gsem, ssem):
        c = lax.axis_index("c")
        s = lax.axis_index("s")
        wid = c * NS + s

        for k in range(nz):
            pltpu.sync_copy(zeros_hbm, acc.at[pl.ds(s * rpt + k * _ZR, _ZR)])
        if tail:
            @pl.when(s == NS - 1)
            def _zero_tail():
                pltpu.sync_copy(zeros_hbm.at[pl.ds(0, tail)],
                                acc.at[pl.ds(NS * rpt, tail)])
        plsc.subcore_barrier()

        def _idx(ch, b):
            return pltpu.make_async_copy(
                sd_hbm.at[wid, ch], idxr.at[b], isem.at[b])

        def _gather(b):
            return pltpu.make_async_copy(
                g_hbm.at[idxr.at[b, 0]], rows.at[b], gsem.at[b])

        def _scatter(b):
            return pltpu.make_async_copy(
                rows.at[b], acc.at[idxr.at[b, 1]], ssem.at[b])

        def _fire_idx(g, B):
            for b in range(_K):
                _idx(g * _K + b, B + b).start()

        def _fire_gather(g, B):
            for b in range(_K):
                _idx(g * _K + b, B + b).wait()
                _gather(B + b).start()

        def _fire_scatter(B):
            for b in range(_K):
                _gather(B + b).wait()
                _scatter(B + b).start(add=True)

        def _drain_scatter(B):
            for b in range(_K):
                _scatter(B + b).wait()

        ngrp = nch // _K

        _fire_idx(0, 0)
        _fire_gather(0, 0)
        _fire_idx(1, _K)
        _fire_scatter(0)

        def _step(g, B, Bo):
            _fire_gather(g, B)
            _drain_scatter(Bo)
            _fire_idx(g + 1, Bo)
            _fire_scatter(B)

        def _body(p, _):
            _step(2 * p + 1, _K, 0)
            _step(2 * p + 2, 0, _K)
            return 0
        lax.fori_loop(0, (ngrp - 2) // 2, _body, 0)

        _fire_gather(ngrp - 1, _K)
        _drain_scatter(0)
        _fire_scatter(_K)
        _drain_scatter(_K)

        plsc.subcore_barrier()
        pltpu.sync_copy(acc.at[pl.ds(s * rpt, rpt)],
                        part_hbm.at[c, pl.ds(s * rpt, rpt)])
        if tail:
            @pl.when(s == NS - 1)
            def _copy_tail():
                pltpu.sync_copy(acc.at[pl.ds(NS * rpt, tail)],
                                part_hbm.at[c, pl.ds(NS * rpt, tail)])

    return agg_kernel


def _scale_mm_body(x_ref, w_ref, d0_ref, d1_ref, out_ref, dis_ref):
    d = d0_ref[...] + d1_ref[...] + 1.0
    dis = lax.rsqrt(d)
    dis_ref[...] = dis
    h = jnp.dot(x_ref[...], w_ref[...], preferred_element_type=jnp.float32)
    out_ref[...] = h * dis


def _mid_body(p_ref, g_ref, dis_ref, b_ref, w_ref, out_ref):
    t = (p_ref[0] + p_ref[1] + g_ref[...]) * dis_ref[...] + b_ref[...]
    o = jnp.maximum(t, 0.0)
    out_ref[...] = jnp.dot(
        o, w_ref[...], preferred_element_type=jnp.float32) * dis_ref[...]


def _final_body(p_ref, g_ref, dis_ref, b_ref, out_ref):
    out_ref[...] = ((p_ref[0] + p_ref[1] + g_ref[...]) * dis_ref[...]
                    + b_ref[...])


def kernel(x, edge_index, W1, b1, W2, b2):
    N, D = x.shape
    E = edge_index.shape[1]
    NP = 10240

    gran = NW * 320
    E_pad = -(-E // gran) * gran
    pad = E_pad - E
    ew = E_pad // NW
    nch = ew // _CK

    src_p = jnp.concatenate(
        [edge_index[0], (jnp.arange(pad, dtype=jnp.int32) % N)])
    dst_p = jnp.concatenate(
        [edge_index[1], N + (jnp.arange(pad, dtype=jnp.int32) % 8)])
    sd = jnp.stack([src_p.reshape(NW, nch, _CK),
                    dst_p.reshape(NW, nch, _CK)], axis=2)
    dst_r = dst_p.reshape(NW, ew // _DCK, _DCK)

    degp = _make_deg_kernel(ew, NP)(dst_r)

    BM = 1000
    grid = (N // BM,)
    row_spec = pl.BlockSpec((BM, D), lambda i: (i, 0))
    dis_spec = pl.BlockSpec((BM, 1), lambda i: (i, 0))
    w_spec = pl.BlockSpec((D, D), lambda i: (0, 0))
    b_spec = pl.BlockSpec((1, D), lambda i: (0, 0))
    p_spec = pl.BlockSpec((NC, BM, D), lambda i: (0, i, 0))
    out_sds = jax.ShapeDtypeStruct((N, D), jnp.float32)

    agg = _make_agg_kernel(N, ew, D)
    zeros_z = jnp.zeros((_ZR, D), jnp.float32)

    g1, dis = pl.pallas_call(
        _scale_mm_body, grid=grid,
        in_specs=[row_spec, w_spec, dis_spec, dis_spec],
        out_specs=[row_spec, dis_spec],
        out_shape=[out_sds, jax.ShapeDtypeStruct((N, 1), jnp.float32)],
    )(x, W1, degp[0].reshape(NP, 1), degp[1].reshape(NP, 1))

    p1 = agg(g1, sd, zeros_z)

    g2 = pl.pallas_call(
        _mid_body, grid=grid,
        in_specs=[p_spec, row_spec, dis_spec, b_spec, w_spec],
        out_specs=row_spec, out_shape=out_sds,
    )(p1, g1, dis, b1.reshape(1, D), W2)

    p2 = agg(g2, sd, zeros_z)

    out = pl.pallas_call(
        _final_body, grid=grid,
        in_specs=[p_spec, row_spec, dis_spec, b_spec],
        out_specs=row_spec, out_shape=out_sds,
    )(p2, g2, dis, b2.reshape(1, D))

    return out
